# baseline (speedup 1.0000x reference)
import functools

import jax
import jax.numpy as jnp
from jax import lax
from jax.experimental import pallas as pl
from jax.experimental.pallas import tpu as pltpu
from jax.experimental.pallas import tpu_sc as plsc

B = 1024
V = 100000
L = 50

NC = 2
NS = 16
NW = NC * NS
PER_W = B * L // NW
G = 20
C = PER_W // G
LANES = 16

_mesh = plsc.VectorSubcoreMesh(core_axis_name="c", subcore_axis_name="s")


@functools.partial(
    pl.kernel,
    out_type=jax.ShapeDtypeStruct((NW, LANES), jnp.float32),
    mesh=_mesh,
    scratch_types=[
        pltpu.VMEM((G, C), jnp.int32),
        pltpu.VMEM((G, C), jnp.float32),
        pltpu.VMEM((G, C), jnp.float32),
        pltpu.VMEM((LANES,), jnp.float32),
        pltpu.SemaphoreType.DMA,
    ],
)
def _pg_gather_mac(idx_hbm, rew_hbm, pred_hbm, out_hbm,
                   idx_v, rew_v, val_v, acc_v, sem):
    wid = lax.axis_index("s") * NC + lax.axis_index("c")
    pltpu.sync_copy(idx_hbm.at[wid], idx_v)
    pltpu.sync_copy(rew_hbm.at[wid], rew_v)
    copies = [
        pltpu.async_copy(pred_hbm.at[idx_v.at[g]], val_v.at[g], sem)
        for g in range(G)
    ]
    for cp in copies:
        cp.wait()
    acc = jnp.zeros((LANES,), jnp.float32)
    for g in range(G):
        for r in range(0, C, LANES):
            acc = acc + val_v[g, pl.ds(r, LANES)] * rew_v[g, pl.ds(r, LANES)]
    acc_v[...] = acc
    pltpu.sync_copy(acc_v, out_hbm.at[wid])


def kernel(pred, target, reward):
    tgt = target.astype(jnp.int32)
    row_off = (jnp.arange(B, dtype=jnp.int32) * V)[:, None]
    flat_idx = (tgt + row_off).reshape(NW, G, C)
    rew = reward.astype(jnp.float32).reshape(NW, G, C)
    partial = _pg_gather_mac(flat_idx, rew, pred.reshape(-1))
    return -jnp.sum(partial) / B


# speedup vs baseline: 30.8659x; 30.8659x over previous
import functools

import jax
import jax.numpy as jnp
from jax import lax
from jax.experimental import pallas as pl
from jax.experimental.pallas import tpu as pltpu
from jax.experimental.pallas import tpu_sc as plsc

B = 1024
V = 100000
L = 50

NC = 2
NS = 16
NW = NC * NS
PER_W = B * L // NW
G = 20
C = PER_W // G
LANES = 16

_mesh = plsc.VectorSubcoreMesh(core_axis_name="c", subcore_axis_name="s")


@functools.partial(
    pl.kernel,
    out_type=jax.ShapeDtypeStruct((NW, LANES), jnp.float32),
    mesh=_mesh,
    scratch_types=[
        pltpu.VMEM((G, C), jnp.int32),
        pltpu.VMEM((G, C), jnp.float32),
        pltpu.VMEM((G, C), jnp.float32),
        pltpu.VMEM((LANES,), jnp.float32),
        pltpu.SemaphoreType.DMA,
    ],
)
def _pg_gather_mac(idx_hbm, rew_hbm, pred_hbm, out_hbm,
                   idx_v, rew_v, val_v, acc_v, sem):
    wid = lax.axis_index("s") * NC + lax.axis_index("c")
    pltpu.sync_copy(idx_hbm.at[wid], idx_v)
    pltpu.sync_copy(rew_hbm.at[wid], rew_v)
    copies = [
        pltpu.async_copy(pred_hbm.at[idx_v.at[g]], val_v.at[g], sem)
        for g in range(G)
    ]
    for cp in copies:
        cp.wait()
    acc = jnp.zeros((LANES,), jnp.float32)
    for g in range(G):
        for r in range(0, C, LANES):
            acc = acc + val_v[g, pl.ds(r, LANES)] * rew_v[g, pl.ds(r, LANES)]
    acc_v[...] = acc
    pltpu.sync_copy(acc_v, out_hbm.at[wid])


def kernel(pred, target, reward):
    t = target.astype(jnp.int32)
    i = jnp.arange(B, dtype=jnp.int32)[:, None]
    n = ((t >> 3) << 13) + ((i >> 7) << 10) + ((t & 7) << 7) + (i & 127)
    flat_idx = n.reshape(NW, G, C)
    rew = reward.astype(jnp.float32).reshape(NW, G, C)
    pred_lin = pred.reshape(8, 128, V // 8, 8).transpose(2, 0, 3, 1).reshape(-1)
    partial = _pg_gather_mac(flat_idx, rew, pred_lin)
    return -jnp.sum(partial) / B


# speedup vs baseline: 32.6940x; 1.0592x over previous
import functools

import jax
import jax.numpy as jnp
from jax import lax
from jax.experimental import pallas as pl
from jax.experimental.pallas import tpu as pltpu
from jax.experimental.pallas import tpu_sc as plsc

B = 1024
V = 100000
L = 50

NC = 2
NS = 16
NW = NC * NS
PER_W = B * L // NW
CHUNK = 128
LANES = 16

_CHUNKS = []
_off = 0
while _off < PER_W:
    _c = min(CHUNK, PER_W - _off)
    _CHUNKS.append((_off, _c))
    _off += _c

_mesh = plsc.VectorSubcoreMesh(core_axis_name="c", subcore_axis_name="s")


@functools.partial(
    pl.kernel,
    out_type=jax.ShapeDtypeStruct((NW, LANES), jnp.float32),
    mesh=_mesh,
    scratch_types=[
        pltpu.VMEM((PER_W,), jnp.int32),
        pltpu.VMEM((PER_W,), jnp.float32),
        pltpu.VMEM((PER_W,), jnp.float32),
        pltpu.VMEM((LANES,), jnp.float32),
        pltpu.SemaphoreType.DMA,
        pltpu.SemaphoreType.DMA,
    ],
)
def _pg_gather_mac(idx_hbm, rew_hbm, pred_hbm, out_hbm,
                   idx_v, rew_v, val_v, acc_v, sem_in, sem_g):
    wid = lax.axis_index("s") * NC + lax.axis_index("c")
    rew_cp = pltpu.async_copy(rew_hbm.at[wid], rew_v, sem_in)
    pltpu.sync_copy(idx_hbm.at[wid], idx_v)
    copies = [
        pltpu.async_copy(pred_hbm.at[idx_v.at[pl.ds(o, c)]],
                         val_v.at[pl.ds(o, c)], sem_g)
        for o, c in _CHUNKS
    ]
    rew_cp.wait()
    acc = jnp.zeros((LANES,), jnp.float32)
    for (o, c), cp in zip(_CHUNKS, copies):
        cp.wait()
        for r in range(o, o + c, LANES):
            acc = acc + val_v[pl.ds(r, LANES)] * rew_v[pl.ds(r, LANES)]
    acc_v[...] = acc
    pltpu.sync_copy(acc_v, out_hbm.at[wid])


def kernel(pred, target, reward):
    t = target.astype(jnp.int32)
    i = jnp.arange(B, dtype=jnp.int32)[:, None]
    n = ((t >> 3) << 13) + ((i >> 7) << 10) + ((t & 7) << 7) + (i & 127)
    flat_idx = n.reshape(NW, PER_W)
    rew = reward.astype(jnp.float32).reshape(NW, PER_W)
    pred_lin = pred.reshape(8, 128, V // 8, 8).transpose(2, 0, 3, 1).reshape(-1)
    partial = _pg_gather_mac(flat_idx, rew, pred_lin)
    return -jnp.sum(partial) / B
